# Initial kernel scaffold; baseline (speedup 1.0000x reference)
import jax
import jax.numpy as jnp
from jax.experimental import pallas as pl


def kernel(x):
    raise NotImplementedError("write your pallas kernel here")


# speedup vs baseline: 2.1207x; 2.1207x over previous
import functools

import jax
import jax.numpy as jnp
import numpy as np
from jax import lax
from jax.experimental import pallas as pl
from jax.experimental.pallas import tpu as pltpu
from jax.experimental.pallas import tpu_sc as plsc

B, C, H, W = 16, 768, 28, 28
D = H * W
R = B * C
NW = 32
ROWS_PER_W = R // NW
CHUNK = 64
NCHUNK = ROWS_PER_W // CHUNK

_PERM = np.array([
    121, 753, 617, 480, 35, 577, 130, 263, 557, 148, 197, 410, 649, 398, 605, 45, 520, 176, 569, 591, 462, 446, 659, 366, 575, 257, 179, 139, 315, 501, 709, 188, 312, 499, 318, 448, 304, 739, 99, 707, 309, 567, 144, 748, 602, 152, 517, 189, 582, 487, 552, 750, 544, 516, 325, 31, 112, 532, 518, 495, 356, 493, 507, 543, 268, 429, 538, 409, 541, 85, 762, 712, 714, 63, 117, 417, 174, 565, 441, 509, 584, 525, 481, 272, 114, 752, 254, 564, 524, 82, 703, 65, 7, 693, 350, 4, 101, 607, 765, 650, 463, 452, 444, 102, 78, 163, 708, 157, 694, 302, 183, 704, 29, 240, 177, 278, 259, 638, 590, 108, 553, 698, 305, 83, 129, 585, 367, 212, 277, 504, 300, 44, 603, 211, 16, 58, 690, 123, 562, 37, 336, 580, 111, 19, 61, 540, 447, 673, 598, 2, 142, 736, 34, 542, 369, 339, 654, 551, 156, 436, 5, 461, 589, 415, 90, 715, 706, 363, 514, 175, 167, 284, 379, 251, 600, 110, 619, 72, 155, 578, 670, 178, 323, 675, 755, 291, 388, 730, 681, 269, 535, 354, 573, 728, 533, 665, 368, 601, 219, 510, 153, 30, 275, 705, 42, 186, 342, 406, 468, 439, 660, 307, 256, 419, 663, 246, 3, 643, 362, 380, 327, 393, 70, 729, 566, 378, 400, 271, 592, 588, 522, 614, 488, 311, 67, 612, 273, 223, 422, 39, 56, 630, 274, 192, 169, 349, 218, 195, 476, 173, 245, 241, 69, 383, 646, 80, 22, 571, 6, 321, 199, 345, 118, 235, 766, 54, 442, 479, 423, 266, 721, 77, 425, 147, 18, 340, 298, 599, 249, 294, 375, 382, 667, 10, 635, 570, 689, 699, 751, 11, 234, 53, 236, 455, 641, 722, 528, 664, 94, 515, 332, 511, 331, 437, 353, 684, 489, 287, 604, 32, 217, 283, 355, 529, 407, 159, 440, 15, 470, 184, 49, 548, 137, 50, 558, 701, 138, 20, 563, 549, 445, 749, 237, 596, 618, 280, 253, 185, 583, 527, 717, 460, 595, 43, 767, 389, 335, 593, 561, 258, 370, 344, 700, 92, 8, 503, 734, 324, 140, 233, 737, 611, 24, 757, 81, 239, 610, 314, 653, 453, 695, 96, 609, 475, 467, 154, 724, 696, 135, 472, 490, 469, 559, 500, 264, 160, 657, 678, 106, 128, 265, 426, 386, 191, 9, 685, 686, 200, 40, 677, 187, 71, 732, 346, 726, 625, 719, 725, 438, 333, 248, 645, 661, 164, 207, 688, 93, 652, 59, 201, 615, 158, 210, 420, 402, 75, 741, 716, 639, 508, 131, 411, 97, 66, 727, 25, 196, 424, 364, 497, 242, 338, 206, 243, 397, 341, 613, 450, 414, 238, 720, 560, 764, 295, 691, 581, 432, 431, 647, 308, 73, 710, 512, 320, 13, 52, 687, 763, 556, 622, 642, 631, 491, 203, 289, 702, 303, 202, 255, 194, 88, 672, 250, 337, 62, 230, 150, 261, 674, 330, 262, 209, 586, 760, 132, 357, 87, 76, 198, 486, 626, 60, 759, 740, 735, 244, 457, 651, 47, 392, 374, 597, 276, 683, 33, 79, 606, 451, 180, 403, 723, 247, 14, 459, 286, 421, 594, 458, 228, 17, 629, 38, 86, 608, 550, 231, 190, 232, 545, 482, 23, 536, 640, 105, 484, 395, 658, 427, 301, 474, 376, 555, 637, 405, 546, 494, 471, 391, 574, 648, 534, 668, 624, 313, 220, 676, 0, 473, 145, 371, 579, 213, 226, 381, 133, 281, 758, 41, 64, 572, 416, 21, 655, 443, 161, 576, 744, 279, 285, 679, 166, 124, 116, 449, 26, 165, 168, 193, 57, 208, 713, 181, 89, 146, 182, 126, 125, 297, 1, 115, 28, 113, 731, 692, 530, 628, 225, 361, 351, 537, 465, 172, 377, 162, 738, 48, 170, 466, 666, 505, 227, 36, 252, 502, 492, 521, 119, 151, 385, 682, 306, 662, 120, 372, 390, 224, 761, 523, 616, 122, 270, 100, 568, 418, 433, 329, 365, 396, 526, 91, 519, 222, 733, 644, 669, 55, 747, 496, 498, 103, 620, 51, 671, 293, 215, 384, 127, 98, 743, 483, 697, 506, 282, 745, 107, 27, 322, 74, 136, 229, 711, 319, 328, 531, 430, 343, 621, 204, 221, 623, 296, 12, 134, 454, 477, 554, 627, 408, 109, 84, 539, 587, 428, 317, 513, 358, 394, 299, 205, 171, 288, 143, 632, 68, 267, 216, 435, 547, 149, 485, 434, 141, 464, 334, 404, 634, 104, 352, 95, 387, 316, 742, 718, 633, 214, 290, 754, 46, 310, 348, 401, 260, 656, 478, 292, 680, 359, 326, 347, 456, 399, 373, 412, 360, 413, 636, 756, 746,
], dtype=np.int32)


def _gather_indices() -> np.ndarray:
    r = np.arange(R)
    idx = (r // C) * C + _PERM[r % C]
    return idx.astype(np.int32).reshape(NW, NCHUNK, CHUNK)


@functools.partial(
    pl.kernel,
    mesh=plsc.VectorSubcoreMesh(core_axis_name="c", subcore_axis_name="s"),
    out_type=jax.ShapeDtypeStruct((R, D), jnp.float32),
    scratch_types=[
        pltpu.VMEM((CHUNK,), jnp.int32),
        pltpu.VMEM((CHUNK, D), jnp.float32),
        pltpu.SemaphoreType.DMA,
    ],
    compiler_params=pltpu.CompilerParams(use_tc_tiling_on_sc=False),
)
def _permute_rows(x_hbm, idx_hbm, out_hbm, idx_v, rows_v, sem):
    nc = lax.axis_size("c")
    wid = lax.axis_index("s") * nc + lax.axis_index("c")
    base = wid * ROWS_PER_W
    for j in range(NCHUNK):
        pltpu.sync_copy(idx_hbm.at[wid, j], idx_v)
        pltpu.async_copy(x_hbm.at[idx_v], rows_v, sem).wait()
        pltpu.sync_copy(rows_v, out_hbm.at[pl.ds(base + j * CHUNK, CHUNK)])


def kernel(x):
    idx = jnp.asarray(_gather_indices())
    out = _permute_rows(x.reshape(R, D), idx)
    return out.reshape(B, C, H, W)


# speedup vs baseline: 2.2156x; 1.0447x over previous
import functools

import jax
import jax.numpy as jnp
import numpy as np
from jax import lax
from jax.experimental import pallas as pl
from jax.experimental.pallas import tpu as pltpu
from jax.experimental.pallas import tpu_sc as plsc

B, C, H, W = 16, 768, 28, 28
R = B * H * W
NW = 32
ROWS_PER_W = R // NW
CHUNK = 56
NCHUNK = ROWS_PER_W // CHUNK
LANES = 16
NCVEC = C // LANES

_PERM = np.array([
    121, 753, 617, 480, 35, 577, 130, 263, 557, 148, 197, 410, 649, 398, 605, 45, 520, 176, 569, 591, 462, 446, 659, 366, 575, 257, 179, 139, 315, 501, 709, 188, 312, 499, 318, 448, 304, 739, 99, 707, 309, 567, 144, 748, 602, 152, 517, 189, 582, 487, 552, 750, 544, 516, 325, 31, 112, 532, 518, 495, 356, 493, 507, 543, 268, 429, 538, 409, 541, 85, 762, 712, 714, 63, 117, 417, 174, 565, 441, 509, 584, 525, 481, 272, 114, 752, 254, 564, 524, 82, 703, 65, 7, 693, 350, 4, 101, 607, 765, 650, 463, 452, 444, 102, 78, 163, 708, 157, 694, 302, 183, 704, 29, 240, 177, 278, 259, 638, 590, 108, 553, 698, 305, 83, 129, 585, 367, 212, 277, 504, 300, 44, 603, 211, 16, 58, 690, 123, 562, 37, 336, 580, 111, 19, 61, 540, 447, 673, 598, 2, 142, 736, 34, 542, 369, 339, 654, 551, 156, 436, 5, 461, 589, 415, 90, 715, 706, 363, 514, 175, 167, 284, 379, 251, 600, 110, 619, 72, 155, 578, 670, 178, 323, 675, 755, 291, 388, 730, 681, 269, 535, 354, 573, 728, 533, 665, 368, 601, 219, 510, 153, 30, 275, 705, 42, 186, 342, 406, 468, 439, 660, 307, 256, 419, 663, 246, 3, 643, 362, 380, 327, 393, 70, 729, 566, 378, 400, 271, 592, 588, 522, 614, 488, 311, 67, 612, 273, 223, 422, 39, 56, 630, 274, 192, 169, 349, 218, 195, 476, 173, 245, 241, 69, 383, 646, 80, 22, 571, 6, 321, 199, 345, 118, 235, 766, 54, 442, 479, 423, 266, 721, 77, 425, 147, 18, 340, 298, 599, 249, 294, 375, 382, 667, 10, 635, 570, 689, 699, 751, 11, 234, 53, 236, 455, 641, 722, 528, 664, 94, 515, 332, 511, 331, 437, 353, 684, 489, 287, 604, 32, 217, 283, 355, 529, 407, 159, 440, 15, 470, 184, 49, 548, 137, 50, 558, 701, 138, 20, 563, 549, 445, 749, 237, 596, 618, 280, 253, 185, 583, 527, 717, 460, 595, 43, 767, 389, 335, 593, 561, 258, 370, 344, 700, 92, 8, 503, 734, 324, 140, 233, 737, 611, 24, 757, 81, 239, 610, 314, 653, 453, 695, 96, 609, 475, 467, 154, 724, 696, 135, 472, 490, 469, 559, 500, 264, 160, 657, 678, 106, 128, 265, 426, 386, 191, 9, 685, 686, 200, 40, 677, 187, 71, 732, 346, 726, 625, 719, 725, 438, 333, 248, 645, 661, 164, 207, 688, 93, 652, 59, 201, 615, 158, 210, 420, 402, 75, 741, 716, 639, 508, 131, 411, 97, 66, 727, 25, 196, 424, 364, 497, 242, 338, 206, 243, 397, 341, 613, 450, 414, 238, 720, 560, 764, 295, 691, 581, 432, 431, 647, 308, 73, 710, 512, 320, 13, 52, 687, 763, 556, 622, 642, 631, 491, 203, 289, 702, 303, 202, 255, 194, 88, 672, 250, 337, 62, 230, 150, 261, 674, 330, 262, 209, 586, 760, 132, 357, 87, 76, 198, 486, 626, 60, 759, 740, 735, 244, 457, 651, 47, 392, 374, 597, 276, 683, 33, 79, 606, 451, 180, 403, 723, 247, 14, 459, 286, 421, 594, 458, 228, 17, 629, 38, 86, 608, 550, 231, 190, 232, 545, 482, 23, 536, 640, 105, 484, 395, 658, 427, 301, 474, 376, 555, 637, 405, 546, 494, 471, 391, 574, 648, 534, 668, 624, 313, 220, 676, 0, 473, 145, 371, 579, 213, 226, 381, 133, 281, 758, 41, 64, 572, 416, 21, 655, 443, 161, 576, 744, 279, 285, 679, 166, 124, 116, 449, 26, 165, 168, 193, 57, 208, 713, 181, 89, 146, 182, 126, 125, 297, 1, 115, 28, 113, 731, 692, 530, 628, 225, 361, 351, 537, 465, 172, 377, 162, 738, 48, 170, 466, 666, 505, 227, 36, 252, 502, 492, 521, 119, 151, 385, 682, 306, 662, 120, 372, 390, 224, 761, 523, 616, 122, 270, 100, 568, 418, 433, 329, 365, 396, 526, 91, 519, 222, 733, 644, 669, 55, 747, 496, 498, 103, 620, 51, 671, 293, 215, 384, 127, 98, 743, 483, 697, 506, 282, 745, 107, 27, 322, 74, 136, 229, 711, 319, 328, 531, 430, 343, 621, 204, 221, 623, 296, 12, 134, 454, 477, 554, 627, 408, 109, 84, 539, 587, 428, 317, 513, 358, 394, 299, 205, 171, 288, 143, 632, 68, 267, 216, 435, 547, 149, 485, 434, 141, 464, 334, 404, 634, 104, 352, 95, 387, 316, 742, 718, 633, 214, 290, 754, 46, 310, 348, 401, 260, 656, 478, 292, 680, 359, 326, 347, 456, 399, 373, 412, 360, 413, 636, 756, 746,
], dtype=np.int32)


@functools.partial(
    pl.kernel,
    mesh=plsc.VectorSubcoreMesh(core_axis_name="c", subcore_axis_name="s"),
    out_type=jax.ShapeDtypeStruct((R, C), jnp.float32),
    scratch_types=[
        pltpu.VMEM((C,), jnp.int32),
        pltpu.VMEM((CHUNK, C), jnp.float32),
        pltpu.VMEM((CHUNK, C), jnp.float32),
    ],
    compiler_params=pltpu.CompilerParams(
        use_tc_tiling_on_sc=True, needs_layout_passes=False
    ),
)
def _permute_channels(x_hbm, perm_hbm, out_hbm, perm_v, in_v, out_v):
    nc = lax.axis_size("c")
    wid = lax.axis_index("s") * nc + lax.axis_index("c")
    base = wid * ROWS_PER_W
    pltpu.sync_copy(perm_hbm, perm_v)
    for j in range(NCHUNK):
        pltpu.sync_copy(x_hbm.at[pl.ds(base + j * CHUNK, CHUNK)], in_v)

        def row_body(r, carry):
            for cc in range(NCVEC):
                idxc = perm_v[pl.ds(cc * LANES, LANES)]
                rows = jnp.full((LANES,), r, jnp.int32)
                out_v[r, pl.ds(cc * LANES, LANES)] = plsc.load_gather(
                    in_v, [rows, idxc]
                )
            return carry

        lax.fori_loop(0, CHUNK, row_body, 0)
        pltpu.sync_copy(out_v, out_hbm.at[pl.ds(base + j * CHUNK, CHUNK)])


def kernel(x):
    xt = jnp.transpose(x, (2, 3, 0, 1)).reshape(R, C)
    out = _permute_channels(xt, jnp.asarray(_PERM))
    return jnp.transpose(out.reshape(H, W, B, C), (2, 3, 0, 1))
